# Initial kernel scaffold; baseline (speedup 1.0000x reference)
import jax
import jax.numpy as jnp
from jax.experimental import pallas as pl


def kernel(x, data_key, ln_g, ln_b, W1, b1, W2, b2, gate_W, gate_b):
    raise NotImplementedError("write your pallas kernel here")


# speedup vs baseline: 138.0862x; 138.0862x over previous
import functools

import jax
import jax.numpy as jnp
from jax.experimental import pallas as pl
from jax.experimental.pallas import tpu as pltpu

_NV = 8
_EPS = 1e-5


def _fused_body(x_ref, ln_g_ref, ln_b_ref, W1_ref, b1_ref, W2_ref, b2_ref,
                gW_ref, gb_ref, out_ref):
    xb = x_ref[...]

    mu = jnp.mean(xb, axis=1, keepdims=True)
    var = jnp.mean((xb - mu) ** 2, axis=1, keepdims=True)
    hn = (xb - mu) * jax.lax.rsqrt(var + _EPS)
    hn = hn * ln_g_ref[...][None, :, :] + ln_b_ref[...][None, :, :]

    s = jnp.mean(hn, axis=0)

    h1 = jax.lax.dot_general(
        W1_ref[...], s, (((0,), (0,)), ((), ())),
        preferred_element_type=jnp.float32,
        precision=jax.lax.Precision.HIGHEST) + b1_ref[...]

    mu2 = jnp.mean(h1, axis=0, keepdims=True)
    var2 = jnp.mean((h1 - mu2) ** 2, axis=0, keepdims=True)
    a = jnp.maximum((h1 - mu2) * jax.lax.rsqrt(var2 + _EPS), 0.0)

    h2 = jax.lax.dot_general(
        W2_ref[...], a, (((0,), (0,)), ((), ())),
        preferred_element_type=jnp.float32,
        precision=jax.lax.Precision.HIGHEST) + b2_ref[...]

    pooled = jnp.mean(xb, axis=2)
    logits = jax.lax.dot_general(
        pooled, gW_ref[...], (((1,), (1,)), ((), ())),
        preferred_element_type=jnp.float32,
        precision=jax.lax.Precision.HIGHEST) + gb_ref[...]
    gate = jax.nn.sigmoid(logits)

    out_ref[...] = xb + gate[:, :, None] * h2[None, :, :]


@jax.jit
def kernel(x, data_key, ln_g, ln_b, W1, b1, W2, b2, gate_W, gate_b):
    B, C, T = x.shape
    n_groups = B // _NV

    grid_spec = pl.GridSpec(
        grid=(n_groups,),
        in_specs=[
            pl.BlockSpec((_NV, C, T), lambda i: (i, 0, 0)),
            pl.BlockSpec((C, 1), lambda i: (0, 0)),
            pl.BlockSpec((C, 1), lambda i: (0, 0)),
            pl.BlockSpec((C, C), lambda i: (0, 0)),
            pl.BlockSpec((C, 1), lambda i: (0, 0)),
            pl.BlockSpec((C, C), lambda i: (0, 0)),
            pl.BlockSpec((C, 1), lambda i: (0, 0)),
            pl.BlockSpec((C, C), lambda i: (0, 0)),
            pl.BlockSpec((1, C), lambda i: (0, 0)),
        ],
        out_specs=pl.BlockSpec((_NV, C, T), lambda i: (i, 0, 0)),
    )

    return pl.pallas_call(
        _fused_body,
        grid_spec=grid_spec,
        out_shape=jax.ShapeDtypeStruct((B, C, T), x.dtype),
        compiler_params=pltpu.CompilerParams(
            dimension_semantics=("arbitrary",),
        ),
    )(x, ln_g.reshape(C, 1), ln_b.reshape(C, 1), W1, b1.reshape(C, 1),
      W2, b2.reshape(C, 1), gate_W, gate_b.reshape(1, C))


# speedup vs baseline: 147.0672x; 1.0650x over previous
import functools

import jax
import jax.numpy as jnp
from jax.experimental import pallas as pl
from jax.experimental.pallas import tpu as pltpu

_NV = 8
_EPS = 1e-5


def _fused_body(x_ref, ln_g_ref, ln_b_ref, W1_ref, b1_ref, W2_ref, b2_ref,
                gW_ref, gb_ref, out_ref):
    xb = x_ref[...]

    mu = jnp.mean(xb, axis=1)
    msq = jnp.mean(xb * xb, axis=1)
    r = jax.lax.rsqrt(msq - mu * mu + _EPS)
    wsum = jnp.sum(xb * r[:, None, :], axis=0)
    corr = jnp.mean(mu * r, axis=0, keepdims=True)
    s = ln_g_ref[...] * (wsum * (1.0 / _NV) - corr) + ln_b_ref[...]

    h1 = jax.lax.dot_general(
        W1_ref[...], s, (((0,), (0,)), ((), ())),
        preferred_element_type=jnp.float32,
        precision=jax.lax.Precision.HIGHEST) + b1_ref[...]

    mu2 = jnp.mean(h1, axis=0, keepdims=True)
    var2 = jnp.mean((h1 - mu2) ** 2, axis=0, keepdims=True)
    a = jnp.maximum((h1 - mu2) * jax.lax.rsqrt(var2 + _EPS), 0.0)

    h2 = jax.lax.dot_general(
        W2_ref[...], a, (((0,), (0,)), ((), ())),
        preferred_element_type=jnp.float32,
        precision=jax.lax.Precision.HIGHEST) + b2_ref[...]

    pooled = jnp.mean(xb, axis=2)
    logits = jax.lax.dot_general(
        pooled, gW_ref[...], (((1,), (1,)), ((), ())),
        preferred_element_type=jnp.float32,
        precision=jax.lax.Precision.HIGHEST) + gb_ref[...]
    gate = jax.nn.sigmoid(logits)

    out_ref[...] = xb + gate[:, :, None] * h2[None, :, :]


@jax.jit
def kernel(x, data_key, ln_g, ln_b, W1, b1, W2, b2, gate_W, gate_b):
    B, C, T = x.shape
    n_groups = B // _NV

    grid_spec = pl.GridSpec(
        grid=(n_groups,),
        in_specs=[
            pl.BlockSpec((_NV, C, T), lambda i: (i, 0, 0)),
            pl.BlockSpec((C, 1), lambda i: (0, 0)),
            pl.BlockSpec((C, 1), lambda i: (0, 0)),
            pl.BlockSpec((C, C), lambda i: (0, 0)),
            pl.BlockSpec((C, 1), lambda i: (0, 0)),
            pl.BlockSpec((C, C), lambda i: (0, 0)),
            pl.BlockSpec((C, 1), lambda i: (0, 0)),
            pl.BlockSpec((C, C), lambda i: (0, 0)),
            pl.BlockSpec((1, C), lambda i: (0, 0)),
        ],
        out_specs=pl.BlockSpec((_NV, C, T), lambda i: (i, 0, 0)),
    )

    return pl.pallas_call(
        _fused_body,
        grid_spec=grid_spec,
        out_shape=jax.ShapeDtypeStruct((B, C, T), x.dtype),
        compiler_params=pltpu.CompilerParams(
            dimension_semantics=("arbitrary",),
        ),
    )(x, ln_g.reshape(C, 1), ln_b.reshape(C, 1), W1, b1.reshape(C, 1),
      W2, b2.reshape(C, 1), gate_W, gate_b.reshape(1, C))


# speedup vs baseline: 159.9304x; 1.0875x over previous
import functools

import jax
import jax.numpy as jnp
from jax.experimental import pallas as pl
from jax.experimental.pallas import tpu as pltpu

_NV = 8
_EPS = 1e-5


def _fused_body(x_ref, ln_g_ref, ln_b_ref, W1_ref, b1_ref, W2_ref, b2_ref,
                gW_ref, gb_ref, out_ref):
    xb = x_ref[...]

    mu = jnp.mean(xb, axis=1)
    msq = jnp.mean(xb * xb, axis=1)
    r = jax.lax.rsqrt(msq - mu * mu + _EPS)
    wsum = jnp.sum(xb * r[:, None, :], axis=0)
    corr = jnp.mean(mu * r, axis=0, keepdims=True)
    s = ln_g_ref[...] * (wsum * (1.0 / _NV) - corr) + ln_b_ref[...]

    h1 = jax.lax.dot_general(
        W1_ref[...], s, (((0,), (0,)), ((), ())),
        preferred_element_type=jnp.float32,
        precision=jax.lax.Precision.DEFAULT) + b1_ref[...]

    mu2 = jnp.mean(h1, axis=0, keepdims=True)
    var2 = jnp.mean((h1 - mu2) ** 2, axis=0, keepdims=True)
    a = jnp.maximum((h1 - mu2) * jax.lax.rsqrt(var2 + _EPS), 0.0)

    h2 = jax.lax.dot_general(
        W2_ref[...], a, (((0,), (0,)), ((), ())),
        preferred_element_type=jnp.float32,
        precision=jax.lax.Precision.DEFAULT) + b2_ref[...]

    pooled = jnp.mean(xb, axis=2)
    logits = jax.lax.dot_general(
        pooled, gW_ref[...], (((1,), (1,)), ((), ())),
        preferred_element_type=jnp.float32,
        precision=jax.lax.Precision.DEFAULT) + gb_ref[...]
    gate = jax.nn.sigmoid(logits)

    out_ref[...] = xb + gate[:, :, None] * h2[None, :, :]


@jax.jit
def kernel(x, data_key, ln_g, ln_b, W1, b1, W2, b2, gate_W, gate_b):
    B, C, T = x.shape
    n_groups = B // _NV

    grid_spec = pl.GridSpec(
        grid=(n_groups,),
        in_specs=[
            pl.BlockSpec((_NV, C, T), lambda i: (i, 0, 0)),
            pl.BlockSpec((C, 1), lambda i: (0, 0)),
            pl.BlockSpec((C, 1), lambda i: (0, 0)),
            pl.BlockSpec((C, C), lambda i: (0, 0)),
            pl.BlockSpec((C, 1), lambda i: (0, 0)),
            pl.BlockSpec((C, C), lambda i: (0, 0)),
            pl.BlockSpec((C, 1), lambda i: (0, 0)),
            pl.BlockSpec((C, C), lambda i: (0, 0)),
            pl.BlockSpec((1, C), lambda i: (0, 0)),
        ],
        out_specs=pl.BlockSpec((_NV, C, T), lambda i: (i, 0, 0)),
    )

    return pl.pallas_call(
        _fused_body,
        grid_spec=grid_spec,
        out_shape=jax.ShapeDtypeStruct((B, C, T), x.dtype),
        compiler_params=pltpu.CompilerParams(
            dimension_semantics=("arbitrary",),
        ),
    )(x, ln_g.reshape(C, 1), ln_b.reshape(C, 1), W1, b1.reshape(C, 1),
      W2, b2.reshape(C, 1), gate_W, gate_b.reshape(1, C))
